# Initial kernel scaffold; baseline (speedup 1.0000x reference)
import jax
import jax.numpy as jnp
from jax.experimental import pallas as pl


def kernel(x, W):
    raise NotImplementedError("write your pallas kernel here")


# speedup vs baseline: 26.2881x; 26.2881x over previous
import functools

import jax
import jax.numpy as jnp
from jax import lax
from jax.experimental import pallas as pl
from jax.experimental.pallas import tpu as pltpu
from jax.experimental.pallas import tpu_sc as plsc

B, L = 4096, 200
V, D = 100000, 2
N = B * L
NC, NS = 2, 16
PER_S = N // NS
NP = 10
P = PER_S // NP
GRP = P // 16


def _make_kernel():
    mesh = plsc.VectorSubcoreMesh(core_axis_name="c", subcore_axis_name="s")

    @functools.partial(
        pl.kernel,
        out_type=jax.ShapeDtypeStruct((NC, NS, NP, P), jnp.float32),
        mesh=mesh,
        scratch_types=[
            pltpu.VMEM((V,), jnp.float32),
            pltpu.VMEM((P,), jnp.int32),
            pltpu.VMEM((P,), jnp.float32),
        ],
        compiler_params=pltpu.CompilerParams(needs_layout_passes=False),
    )
    def k(x_hbm, wt_hbm, out_hbm, col_v, idx_v, val_v):
        c = lax.axis_index("c")
        s = lax.axis_index("s")
        pltpu.sync_copy(wt_hbm.at[c], col_v)

        def gather_group(i, _):
            o = i * 16
            iv = idx_v[pl.ds(o, 16)]
            val_v[pl.ds(o, 16)] = plsc.load_gather(col_v, [iv])
            return 0

        for p in range(NP):
            pltpu.sync_copy(x_hbm.at[s, p], idx_v)
            lax.fori_loop(0, GRP, gather_group, 0)
            pltpu.sync_copy(val_v, out_hbm.at[c, s, p])

    return k


_gather = _make_kernel()


def kernel(x, W):
    xr = x.reshape(NS, NP, P)
    wt = W.T.reshape(NC, V)
    out = _gather(xr, wt)
    return out.reshape(NC, N).T.reshape(B, L, D)


# speedup vs baseline: 27.7074x; 1.0540x over previous
import functools

import jax
import jax.numpy as jnp
from jax import lax
from jax.experimental import pallas as pl
from jax.experimental.pallas import tpu as pltpu
from jax.experimental.pallas import tpu_sc as plsc

B, L = 4096, 200
V, D = 100000, 2
N = B * L
NC, NS = 2, 16
PER_S = N // NS
NP = 10
P = PER_S // NP
GRP = P // 16


def _make_kernel():
    mesh = plsc.VectorSubcoreMesh(core_axis_name="c", subcore_axis_name="s")

    @functools.partial(
        pl.kernel,
        out_type=jax.ShapeDtypeStruct((NC, NS, NP, P), jnp.float32),
        mesh=mesh,
        scratch_types=[
            pltpu.VMEM((V,), jnp.float32),
            pltpu.VMEM((P,), jnp.int32),
            pltpu.VMEM((P,), jnp.float32),
        ],
        compiler_params=pltpu.CompilerParams(needs_layout_passes=False),
    )
    def k(x_hbm, wt_hbm, out_hbm, col_v, idx_v, val_v):
        c = lax.axis_index("c")
        s = lax.axis_index("s")
        pltpu.sync_copy(wt_hbm.at[c], col_v)

        def gather_block(i, _):
            for u in range(8):
                o = i * 128 + u * 16
                val_v[pl.ds(o, 16)] = plsc.load_gather(col_v, [idx_v[pl.ds(o, 16)]])
            return 0

        for p in range(NP):
            pltpu.sync_copy(x_hbm.at[s, p], idx_v)
            lax.fori_loop(0, GRP // 8, gather_block, 0)
            pltpu.sync_copy(val_v, out_hbm.at[c, s, p])

    return k


_gather = _make_kernel()


def kernel(x, W):
    xr = x.reshape(NS, NP, P)
    wt = W.T.reshape(NC, V)
    out = _gather(xr, wt)
    return out.reshape(NC, N).T.reshape(B, L, D)


# speedup vs baseline: 31.1816x; 1.1254x over previous
import functools

import jax
import jax.numpy as jnp
from jax import lax
from jax.experimental import pallas as pl
from jax.experimental.pallas import tpu as pltpu
from jax.experimental.pallas import tpu_sc as plsc

B, L = 4096, 200
V, D = 100000, 2
N = B * L
NC, NS = 2, 16
PER_S = N // NS
NP = 10
P = PER_S // NP
GRP = P // 16


def _make_kernel():
    mesh = plsc.VectorSubcoreMesh(core_axis_name="c", subcore_axis_name="s")

    @functools.partial(
        pl.kernel,
        out_type=jax.ShapeDtypeStruct((NC, NS, NP, P), jnp.float32),
        mesh=mesh,
        scratch_types=[
            pltpu.VMEM((V,), jnp.float32),
            pltpu.VMEM((P,), jnp.int32),
            pltpu.VMEM((P,), jnp.int32),
            pltpu.VMEM((P,), jnp.float32),
            pltpu.VMEM((P,), jnp.float32),
            pltpu.SemaphoreType.DMA,
            pltpu.SemaphoreType.DMA,
        ],
        compiler_params=pltpu.CompilerParams(needs_layout_passes=False),
    )
    def k(x_hbm, wt_hbm, out_hbm, col_v, idx0, idx1, val0, val1, sem_i, sem_o):
        c = lax.axis_index("c")
        s = lax.axis_index("s")
        idx_bufs = (idx0, idx1)
        val_bufs = (val0, val1)
        first_idx = pltpu.async_copy(x_hbm.at[s, 0], idx0, sem_i)
        pltpu.sync_copy(wt_hbm.at[c], col_v)

        idx_cps = [first_idx]
        out_cps = []
        for p in range(NP):
            cur = p % 2
            if p + 1 < NP:
                idx_cps.append(
                    pltpu.async_copy(x_hbm.at[s, p + 1], idx_bufs[1 - cur], sem_i)
                )
            idx_cps[p].wait()
            if p >= 2:
                out_cps[p - 2].wait()

            ib = idx_bufs[cur]
            vb = val_bufs[cur]

            def gather_block(i, _):
                for u in range(8):
                    o = i * 128 + u * 16
                    vb[pl.ds(o, 16)] = plsc.load_gather(col_v, [ib[pl.ds(o, 16)]])
                return 0

            lax.fori_loop(0, GRP // 8, gather_block, 0)
            out_cps.append(pltpu.async_copy(vb, out_hbm.at[c, s, p], sem_o))
        out_cps[NP - 2].wait()
        out_cps[NP - 1].wait()

    return k


_gather = _make_kernel()


def kernel(x, W):
    xr = x.reshape(NS, NP, P)
    wt = W.T.reshape(NC, V)
    out = _gather(xr, wt)
    return out.reshape(NC, N).T.reshape(B, L, D)


# speedup vs baseline: 35.8525x; 1.1498x over previous
import functools

import jax
import jax.numpy as jnp
from jax import lax
from jax.experimental import pallas as pl
from jax.experimental.pallas import tpu as pltpu
from jax.experimental.pallas import tpu_sc as plsc

B, L = 4096, 200
V, D = 100000, 2
N = B * L
NC, NS = 2, 16
PER_S = N // NS
NP = 10
P = PER_S // NP
GRP = P // 16


def _make_kernel():
    mesh = plsc.VectorSubcoreMesh(core_axis_name="c", subcore_axis_name="s")

    @functools.partial(
        pl.kernel,
        out_type=jax.ShapeDtypeStruct((NC, N), jnp.float32),
        mesh=mesh,
        scratch_types=[
            pltpu.VMEM((V,), jnp.float32),
            pltpu.VMEM((P,), jnp.int32),
            pltpu.VMEM((P,), jnp.int32),
            pltpu.VMEM((P,), jnp.float32),
            pltpu.VMEM((P,), jnp.float32),
            pltpu.SemaphoreType.DMA,
            pltpu.SemaphoreType.DMA,
        ],
        compiler_params=pltpu.CompilerParams(needs_layout_passes=False),
    )
    def k(x_hbm, wt_hbm, out_hbm, col_v, idx0, idx1, val0, val1, sem_i, sem_o):
        c = lax.axis_index("c")
        s = lax.axis_index("s")
        idx_bufs = (idx0, idx1)
        val_bufs = (val0, val1)
        first_idx = pltpu.async_copy(x_hbm.at[s, 0], idx0, sem_i)
        pltpu.sync_copy(wt_hbm.at[c], col_v)

        idx_cps = [first_idx]
        out_cps = []
        for p in range(NP):
            cur = p % 2
            if p + 1 < NP:
                idx_cps.append(
                    pltpu.async_copy(x_hbm.at[s, p + 1], idx_bufs[1 - cur], sem_i)
                )
            idx_cps[p].wait()
            if p >= 2:
                out_cps[p - 2].wait()

            ib = idx_bufs[cur]
            vb = val_bufs[cur]

            def gather_block(i, _):
                for u in range(8):
                    o = i * 128 + u * 16
                    vb[pl.ds(o, 16)] = plsc.load_gather(col_v, [ib[pl.ds(o, 16)]])
                return 0

            lax.fori_loop(0, GRP // 8, gather_block, 0)
            base = s * PER_S + p * P
            out_cps.append(
                pltpu.async_copy(vb, out_hbm.at[c, pl.ds(base, P)], sem_o)
            )
        out_cps[NP - 2].wait()
        out_cps[NP - 1].wait()

    return k


_gather = _make_kernel()


def kernel(x, W):
    xr = x.reshape(NS, NP, P)
    wt = W.T.reshape(NC, V)
    out = _gather(xr, wt)
    return out.T.reshape(B, L, D)
